# baseline (speedup 1.0000x reference)
import functools

import jax
import jax.numpy as jnp
from jax import lax
from jax.experimental import pallas as pl
from jax.experimental.pallas import tpu as pltpu
from jax.experimental.pallas import tpu_sc as plsc

VOCAB = 1000
VPAD = 1024
NSL = VPAD // 128
B, L = 1024, 20
TOK = B * L

NC, NS = 2, 16
NW = NC * NS
BPW = TOK // NW
TCH = 16
NCH = BPW // TCH
DEPTH = 3
LANES = 16

BLK = 256
NBLK = TOK // BLK


def _sc_gather_body(table_hbm, tflat_hbm, x_hbm, y_hbm, out_hbm, part_hbm,
                    x_v, y_v, idx_v, rows, pidx_v, pval_v, pick_v,
                    gsems, psem):
    wid = lax.axis_index("s") * NC + lax.axis_index("c")
    base = pl.multiple_of(wid * BPW, BPW)
    obase = pl.multiple_of(wid * (BPW * NSL), BPW * NSL)

    pltpu.sync_copy(x_hbm.at[pl.ds(base, BPW)], x_v)
    pltpu.sync_copy(y_hbm.at[pl.ds(base, BPW)], y_v)

    for g in range(BPW * NSL // LANES):
        gl = lax.iota(jnp.int32, LANES) + jnp.int32(g * LANES)
        toks = plsc.load_gather(x_v, [lax.shift_right_logical(gl, 3)])
        idx_v[pl.ds(g * LANES, LANES)] = (
            toks * NSL + lax.bitwise_and(gl, NSL - 1))

    cps = [None] * DEPTH

    def fire(k):
        sl = k % DEPTH
        cps[sl] = pltpu.async_copy(
            table_hbm.at[idx_v.at[pl.ds(k * TCH * NSL, TCH * NSL)]],
            rows[sl], gsems[sl])

    for d in range(DEPTH):
        fire(d)
    for k in range(NCH):
        sl = k % DEPTH
        cps[sl].wait()
        pltpu.sync_copy(rows[sl],
                        out_hbm.at[pl.ds(obase + k * (TCH * NSL), TCH * NSL)])
        if k + DEPTH < NCH:
            fire(k + DEPTH)

    acc = jnp.zeros((LANES,), jnp.float32)
    for c in range(BPW // 128):
        for q in range(128 // LANES):
            o = c * 128 + q * LANES
            x16 = x_v[pl.ds(o, LANES)]
            y16 = y_v[pl.ds(o, LANES)]
            pidx_v[pl.ds(q * LANES, LANES)] = x16 * VOCAB + y16
        pltpu.async_copy(tflat_hbm.at[pidx_v], pval_v, psem).wait()
        for q in range(128 // LANES):
            acc = acc + pval_v[pl.ds(q * LANES, LANES)]

    pick_v[...] = acc
    pltpu.sync_copy(pick_v, part_hbm.at[wid])


_sc_gather = functools.partial(
    pl.kernel,
    out_type=[
        jax.ShapeDtypeStruct((TOK * NSL, 128), jnp.float32),
        jax.ShapeDtypeStruct((NW, LANES), jnp.float32),
    ],
    mesh=plsc.VectorSubcoreMesh(
        core_axis_name="c", subcore_axis_name="s",
        num_cores=NC, num_subcores=NS),
    compiler_params=pltpu.CompilerParams(
        use_tc_tiling_on_sc=False, needs_layout_passes=False),
    scratch_types=[
        pltpu.VMEM((BPW,), jnp.int32),
        pltpu.VMEM((BPW,), jnp.int32),
        pltpu.VMEM((BPW * NSL,), jnp.int32),
        [pltpu.VMEM((TCH * NSL, 128), jnp.float32)] * DEPTH,
        pltpu.VMEM((128,), jnp.int32),
        pltpu.VMEM((128,), jnp.float32),
        pltpu.VMEM((LANES,), jnp.float32),
        [pltpu.SemaphoreType.DMA] * DEPTH,
        pltpu.SemaphoreType.DMA,
    ],
)(_sc_gather_body)


def _lse_body(x_ref, table_ref, out_ref, lse_scr, acc_scr):
    pid = pl.program_id(0)
    t = table_ref[...]

    @pl.when(pid == 0)
    def _init():
        m = jnp.max(t, axis=1, keepdims=True)
        s = jnp.sum(jnp.exp(t - m), axis=1, keepdims=True)
        lse_scr[...] = m + jnp.log(s)
        acc_scr[0] = 0.0

    xv = x_ref[0]
    iota = lax.broadcasted_iota(jnp.int32, (BLK, VOCAB), 1)
    oh_x = (xv == iota).astype(jnp.float32)
    lse_tok = lax.dot_general(
        oh_x, lse_scr[...], (((1,), (0,)), ((), ())),
        preferred_element_type=jnp.float32)
    acc_scr[0] += jnp.sum(lse_tok)

    @pl.when(pid == NBLK - 1)
    def _fin():
        out_ref[...] = jnp.full((1, 1), acc_scr[0], jnp.float32)


def _lse_sum(Xr, table):
    return pl.pallas_call(
        _lse_body,
        grid=(NBLK,),
        in_specs=[
            pl.BlockSpec((1, BLK, 1), lambda i: (i, 0, 0)),
            pl.BlockSpec((VOCAB, VOCAB), lambda i: (0, 0)),
        ],
        out_specs=pl.BlockSpec((1, 1), lambda i: (0, 0)),
        out_shape=jax.ShapeDtypeStruct((1, 1), jnp.float32),
        scratch_shapes=[
            pltpu.VMEM((VOCAB, 1), jnp.float32),
            pltpu.SMEM((1,), jnp.float32),
        ],
    )(Xr, table)


def kernel(X, Y, table):
    Xf = X.astype(jnp.int32).reshape(TOK)
    Yf = Y.astype(jnp.int32).reshape(TOK)
    table8 = (jnp.pad(table, ((0, 0), (0, VPAD - VOCAB)))
              .reshape(VOCAB * NSL, 128))
    staged, parts = _sc_gather(table8, table.reshape(VOCAB * VOCAB), Xf, Yf)
    lse_sum = _lse_sum(X.astype(jnp.int32).reshape(NBLK, BLK, 1), table)
    loss = (lse_sum[0, 0] - jnp.sum(parts)) / TOK
    logits = staged.reshape(TOK, VPAD)[:, :VOCAB].reshape(B, L, VOCAB)
    return logits, loss
